# baseline (speedup 1.0000x reference)
import functools
import math

import jax
import jax.numpy as jnp
from jax import lax
from jax.experimental import pallas as pl
from jax.experimental.pallas import tpu as pltpu
from jax.experimental.pallas import tpu_sc as plsc

D = 64
S = 200
LANES = 16
NC, NS = 2, 16
NW = NC * NS

SCALE = math.sqrt(D)

C = 800
M = C // S
SUB = 128
N_SUB = C // SUB
SUB_TAIL = C - N_SUB * SUB


def _emb_body(n_rows, x_hbm, table_hbm, pe_hbm, out_hbm, idx_v, pe_v, buf, sem):
    rows_per_w = n_rows // NW
    n_chunks = rows_per_w // C
    wid = lax.axis_index("s") * NC + lax.axis_index("c")

    pltpu.sync_copy(pe_hbm, pe_v)

    def chunk_body(g, carry):
        base = wid * rows_per_w + g * C

        pltpu.sync_copy(x_hbm.at[pl.ds(base, C)], idx_v)

        for j in range(N_SUB):
            pltpu.async_copy(
                table_hbm.at[idx_v.at[pl.ds(j * SUB, SUB)]],
                buf.at[pl.ds(j * SUB, SUB)],
                sem,
            ).wait()
        if SUB_TAIL:
            pltpu.async_copy(
                table_hbm.at[idx_v.at[pl.ds(N_SUB * SUB, SUB_TAIL)]],
                buf.at[pl.ds(N_SUB * SUB, SUB_TAIL)],
                sem,
            ).wait()

        def s_body(s, carry2):
            for k in range(D // LANES):
                p = pe_v[s, pl.ds(k * LANES, LANES)]
                for r in range(M):
                    row = r * S + s
                    buf[row, pl.ds(k * LANES, LANES)] = (
                        buf[row, pl.ds(k * LANES, LANES)] * SCALE + p
                    )
            return carry2

        lax.fori_loop(0, S, s_body, 0)

        pltpu.sync_copy(buf, out_hbm.at[pl.ds(base, C)])
        return carry

    lax.fori_loop(0, n_chunks, chunk_body, 0)


def kernel(x, table, pe):
    B, seq = x.shape
    n_rows = B * seq
    x_flat = x.reshape(n_rows).astype(jnp.int32)
    pe_s = pe[:seq]

    mesh = plsc.VectorSubcoreMesh(core_axis_name="c", subcore_axis_name="s")
    run = pl.kernel(
        functools.partial(_emb_body, n_rows),
        out_type=jax.ShapeDtypeStruct((n_rows, D), jnp.float32),
        mesh=mesh,
        scratch_types=[
            pltpu.VMEM((C,), jnp.int32),
            pltpu.VMEM((S, D), jnp.float32),
            pltpu.VMEM((C, D), jnp.float32),
            pltpu.SemaphoreType.DMA,
        ],
        compiler_params=pltpu.CompilerParams(use_tc_tiling_on_sc=False),
    )
    out = run(x_flat, table, pe_s)
    return out.reshape(B, seq, D)


# speedup vs baseline: 1.1698x; 1.1698x over previous
import functools
import math

import jax
import jax.numpy as jnp
from jax import lax
from jax.experimental import pallas as pl
from jax.experimental.pallas import tpu as pltpu
from jax.experimental.pallas import tpu_sc as plsc

D = 64
S = 200
LANES = 16
NC, NS = 2, 16
NW = NC * NS

SCALE = math.sqrt(D)

C = 800
M = C // S
SUB = 128
N_SUB = C // SUB
SUB_TAIL = C - N_SUB * SUB


def _fire_gathers(table_hbm, idx, buf, sem):
    for j in range(N_SUB):
        pltpu.async_copy(
            table_hbm.at[idx.at[pl.ds(j * SUB, SUB)]],
            buf.at[pl.ds(j * SUB, SUB)],
            sem,
        )
    if SUB_TAIL:
        pltpu.async_copy(
            table_hbm.at[idx.at[pl.ds(N_SUB * SUB, SUB_TAIL)]],
            buf.at[pl.ds(N_SUB * SUB, SUB_TAIL)],
            sem,
        )


def _emb_body(n_rows, x_hbm, table_hbm, pe_hbm, out_hbm,
              idx0, idx1, pe_v, buf0, buf1,
              semg0, semg1, sems0, sems1, semi0, semi1):
    rows_per_w = n_rows // NW
    n_chunks = rows_per_w // C
    wid = lax.axis_index("s") * NC + lax.axis_index("c")
    w_base = wid * rows_per_w

    idx = (idx0, idx1)
    buf = (buf0, buf1)
    semg = (semg0, semg1)
    sems = (sems0, sems1)
    semi = (semi0, semi1)

    pltpu.sync_copy(pe_hbm, pe_v)

    def compute(b):
        def s_body(s, carry):
            for k in range(D // LANES):
                p = pe_v[s, pl.ds(k * LANES, LANES)]
                for r in range(M):
                    row = r * S + s
                    buf[b][row, pl.ds(k * LANES, LANES)] = (
                        buf[b][row, pl.ds(k * LANES, LANES)] * SCALE + p
                    )
            return carry
        lax.fori_loop(0, S, s_body, 0)

    pltpu.sync_copy(x_hbm.at[pl.ds(w_base, C)], idx[0])
    _fire_gathers(table_hbm, idx[0], buf[0], semg[0])
    pltpu.async_copy(x_hbm.at[pl.ds(w_base + C, C)], idx[1], semi[1])

    def body_for(b, g):
        o = 1 - b
        base = w_base + g * C

        @pl.when(g >= 1)
        def _():
            pltpu.make_async_copy(
                buf[o], out_hbm.at[pl.ds(base - C, C)], sems[o]).wait()

        @pl.when(g + 1 < n_chunks)
        def _():
            pltpu.make_async_copy(
                x_hbm.at[pl.ds(base + C, C)], idx[o], semi[o]).wait()
            _fire_gathers(table_hbm, idx[o], buf[o], semg[o])

        pltpu.make_async_copy(
            out_hbm.at[pl.ds(0, C)], buf[b], semg[b]).wait()

        @pl.when(g + 2 < n_chunks)
        def _():
            pltpu.async_copy(
                x_hbm.at[pl.ds(base + 2 * C, C)], idx[b], semi[b])

        compute(b)

        pltpu.async_copy(buf[b], out_hbm.at[pl.ds(base, C)], sems[b])

    def outer(o_idx, carry):
        body_for(0, o_idx * 2)
        body_for(1, o_idx * 2 + 1)
        return carry

    lax.fori_loop(0, n_chunks // 2, outer, 0)

    last = n_chunks - 1
    pltpu.make_async_copy(
        buf[last % 2],
        out_hbm.at[pl.ds(w_base + last * C, C)],
        sems[last % 2]).wait()


def kernel(x, table, pe):
    B, seq = x.shape
    n_rows = B * seq
    x_flat = x.reshape(n_rows).astype(jnp.int32)
    pe_s = pe[:seq]

    mesh = plsc.VectorSubcoreMesh(core_axis_name="c", subcore_axis_name="s")
    run = pl.kernel(
        functools.partial(_emb_body, n_rows),
        out_type=jax.ShapeDtypeStruct((n_rows, D), jnp.float32),
        mesh=mesh,
        scratch_types=[
            pltpu.VMEM((C,), jnp.int32),
            pltpu.VMEM((C,), jnp.int32),
            pltpu.VMEM((S, D), jnp.float32),
            pltpu.VMEM((C, D), jnp.float32),
            pltpu.VMEM((C, D), jnp.float32),
            pltpu.SemaphoreType.DMA,
            pltpu.SemaphoreType.DMA,
            pltpu.SemaphoreType.DMA,
            pltpu.SemaphoreType.DMA,
            pltpu.SemaphoreType.DMA,
            pltpu.SemaphoreType.DMA,
        ],
        compiler_params=pltpu.CompilerParams(use_tc_tiling_on_sc=False),
    )
    out = run(x_flat, table, pe_s)
    return out.reshape(B, seq, D)
